# baseline (speedup 1.0000x reference)
import functools

import jax
import jax.numpy as jnp
from jax import lax
from jax.experimental import pallas as pl
from jax.experimental.pallas import tpu as pltpu
from jax.experimental.pallas import tpu_sc as plsc

N_NODES = 10000
N_EDGES = 320000
IN_DIM = 128
OUT_DIM = 16
HEADS = 8
HC = HEADS * OUT_DIM

NC = 2
NS = 16
L = 16

EPT = N_EDGES // (NC * NS)
EB = 80
NBLK = EPT // EB
HEXT = 144
DEN_W = 16
NPAD = 10240
RPT = NPAD // NS
RCH = 64

BN1 = 2000


def _proj_body(x_ref, wt_ref, atts_ref, attd_ref, h_ref, as_ref, ad_ref):
    h = jnp.dot(x_ref[...], wt_ref[...], preferred_element_type=jnp.float32)
    h_ref[...] = h
    col = lax.broadcasted_iota(jnp.int32, (HC, HEADS), 0)
    row = lax.broadcasted_iota(jnp.int32, (HC, HEADS), 1)
    m = jnp.where(col // OUT_DIM == row, 1.0, 0.0).astype(jnp.float32)
    as_ref[...] = jnp.dot(h * atts_ref[...], m, preferred_element_type=jnp.float32)
    ad_ref[...] = jnp.dot(h * attd_ref[...], m, preferred_element_type=jnp.float32)


def _proj(x, wt, atts, attd):
    grid = (N_NODES // BN1,)
    return pl.pallas_call(
        _proj_body,
        grid=grid,
        in_specs=[
            pl.BlockSpec((BN1, IN_DIM), lambda i: (i, 0)),
            pl.BlockSpec((IN_DIM, HC), lambda i: (0, 0)),
            pl.BlockSpec((1, HC), lambda i: (0, 0)),
            pl.BlockSpec((1, HC), lambda i: (0, 0)),
        ],
        out_specs=[
            pl.BlockSpec((BN1, HC), lambda i: (i, 0)),
            pl.BlockSpec((BN1, HEADS), lambda i: (i, 0)),
            pl.BlockSpec((BN1, HEADS), lambda i: (i, 0)),
        ],
        out_shape=[
            jax.ShapeDtypeStruct((N_NODES, HC), jnp.float32),
            jax.ShapeDtypeStruct((N_NODES, HEADS), jnp.float32),
            jax.ShapeDtypeStruct((N_NODES, HEADS), jnp.float32),
        ],
    )(x, wt, atts, attd)


def _make_edge_kernel():
    mesh = plsc.VectorSubcoreMesh(core_axis_name="c", subcore_axis_name="s")

    @functools.partial(
        pl.kernel,
        out_type=jax.ShapeDtypeStruct((NC, NPAD, HEXT), jnp.float32),
        mesh=mesh,
        compiler_params=pltpu.CompilerParams(use_tc_tiling_on_sc=False),
        scratch_types=[
            pltpu.VMEM((2, EB), jnp.int32),
            pltpu.VMEM((2, EB), jnp.int32),
            pltpu.VMEM((EB, HEXT), jnp.float32),
            pltpu.VMEM((EB, HEXT), jnp.float32),
            pltpu.VMEM((EB, DEN_W), jnp.float32),
            pltpu.VMEM((EB, DEN_W), jnp.float32),
            pltpu.VMEM((EB, HEXT), jnp.float32),
            pltpu.VMEM_SHARED((NPAD, HEXT), jnp.float32),
            pltpu.SemaphoreType.DMA,
            pltpu.SemaphoreType.DMA,
            pltpu.SemaphoreType.DMA,
            pltpu.SemaphoreType.DMA,
        ],
    )
    def edge_kernel(hext_hbm, adp_hbm, sd_hbm, acc_hbm,
                    sd0, sd1, hg0, hg1, adg0, adg1, msg_v, acc,
                    sh0, sh1, sa0, sa1):
        c = lax.axis_index("c")
        s = lax.axis_index("s")
        wid = c * NS + s
        slots = ((sd0, hg0, adg0, sh0, sa0), (sd1, hg1, adg1, sh1, sa1))

        def _zero(i, _):
            r = i // (HEXT // L)
            col = (i % (HEXT // L)) * L
            msg_v[r, pl.ds(col, L)] = jnp.zeros((L,), jnp.float32)
            return 0

        lax.fori_loop(0, RCH * (HEXT // L), _zero, 0)

        def _init_chunk(k, _):
            r0 = s * RPT + k * RCH
            pltpu.sync_copy(msg_v.at[pl.ds(0, RCH)], acc.at[pl.ds(r0, RCH)])
            return 0

        lax.fori_loop(0, RPT // RCH, _init_chunk, 0)
        plsc.subcore_barrier()

        def _fire(blk, slot):
            sd, hg, adg, sh, sa = slot
            pltpu.sync_copy(sd_hbm.at[:, wid, blk], sd)
            pltpu.async_copy(hext_hbm.at[sd.at[0]], hg, sh)
            pltpu.async_copy(adp_hbm.at[sd.at[1]], adg, sa)

        def _process(slot):
            sd, hg, adg, sh, sa = slot
            pltpu.make_async_copy(hext_hbm.at[sd.at[0]], hg, sh).wait()
            pltpu.make_async_copy(adp_hbm.at[sd.at[1]], adg, sa).wait()

            def _edge(e, _):
                a = hg[e, pl.ds(HC, L)] + adg[e, :]
                w = jnp.exp(jnp.maximum(a, 0.2 * a))
                msg_v[e, pl.ds(HC, L)] = w
                for hh in range(HEADS):
                    msg_v[e, pl.ds(hh * L, L)] = hg[e, pl.ds(hh * L, L)] * w[hh]
                return 0

            lax.fori_loop(0, EB, _edge, 0, unroll=4)
            pltpu.sync_copy(msg_v, acc.at[sd.at[1]], add=True)

        _fire(0, slots[0])

        def _pair(i, _):
            _fire(2 * i + 1, slots[1])
            _process(slots[0])
            _fire(2 * i + 2, slots[0])
            _process(slots[1])
            return 0

        lax.fori_loop(0, (NBLK - 1) // 2, _pair, 0)
        _process(slots[0])
        plsc.subcore_barrier()

        def _drain_chunk(k, _):
            r0 = s * RPT + k * RCH
            pltpu.sync_copy(acc.at[pl.ds(r0, RCH)], msg_v.at[pl.ds(0, RCH)])
            pltpu.sync_copy(msg_v.at[pl.ds(0, RCH)], acc_hbm.at[c, pl.ds(r0, RCH)])
            return 0

        lax.fori_loop(0, RPT // RCH, _drain_chunk, 0)

    return edge_kernel


_edge_kernel = _make_edge_kernel()


def _combine_body(acc_ref, x_ref, bias_ref, out_ref):
    num = acc_ref[0, :, 0:HC] + acc_ref[1, :, 0:HC]
    den = acc_ref[0, :, HC:HC + HEADS] + acc_ref[1, :, HC:HC + HEADS]
    inv = 1.0 / jnp.maximum(den, 1e-30)
    col = lax.broadcasted_iota(jnp.int32, (HEADS, HC), 1)
    row = lax.broadcasted_iota(jnp.int32, (HEADS, HC), 0)
    m = jnp.where(col // OUT_DIM == row, 1.0, 0.0).astype(jnp.float32)
    inv128 = jnp.dot(inv, m, preferred_element_type=jnp.float32)
    out = jnp.maximum(num * inv128 + bias_ref[...], 0.0)
    out_ref[...] = x_ref[...] + out


def _combine(acc2, x, bias2):
    grid = (N_NODES // BN1,)
    return pl.pallas_call(
        _combine_body,
        grid=grid,
        in_specs=[
            pl.BlockSpec((NC, BN1, HEXT), lambda i: (0, i, 0)),
            pl.BlockSpec((BN1, IN_DIM), lambda i: (i, 0)),
            pl.BlockSpec((1, HC), lambda i: (0, 0)),
        ],
        out_specs=pl.BlockSpec((BN1, HC), lambda i: (i, 0)),
        out_shape=jax.ShapeDtypeStruct((N_NODES, HC), jnp.float32),
    )(acc2, x, bias2)


def kernel(x, edge_index, W, att_src, att_dst, bias):
    src = edge_index[0].astype(jnp.int32).reshape(NC * NS, NBLK, EB)
    dst = edge_index[1].astype(jnp.int32).reshape(NC * NS, NBLK, EB)
    sd = jnp.stack([src, dst])
    wt = W.T
    atts = att_src.reshape(1, HC)
    attd = att_dst.reshape(1, HC)
    h, a_s, a_d = _proj(x, wt, atts, attd)
    pad8 = jnp.zeros((N_NODES, 8), jnp.float32)
    hext = jnp.concatenate([h, a_s, pad8], axis=1)
    adp = jnp.concatenate([a_d, pad8], axis=1)
    acc2 = _edge_kernel(hext, adp, sd)
    acc2 = acc2[:, :N_NODES]
    return _combine(acc2, x, bias.reshape(1, HC))


# speedup vs baseline: 1.7372x; 1.7372x over previous
import functools

import jax
import jax.numpy as jnp
from jax import lax
from jax.experimental import pallas as pl
from jax.experimental.pallas import tpu as pltpu
from jax.experimental.pallas import tpu_sc as plsc

N_NODES = 10000
N_EDGES = 320000
IN_DIM = 128
OUT_DIM = 16
HEADS = 8
HC = HEADS * OUT_DIM

NC = 2
NS = 16
L = 16

EPT = N_EDGES // (NC * NS)
EB = 80
SB = EB // 2
NBLK = EPT // EB
HEXT = 144
DEN_W = 16
NPAD = 10240
RPT = NPAD // NS
RCH = 64

BN1 = 2000


def _proj_body(x_ref, wt_ref, atts_ref, attd_ref, h_ref, as_ref, ad_ref):
    h = jnp.dot(x_ref[...], wt_ref[...], preferred_element_type=jnp.float32)
    h_ref[...] = h
    col = lax.broadcasted_iota(jnp.int32, (HC, HEADS), 0)
    row = lax.broadcasted_iota(jnp.int32, (HC, HEADS), 1)
    m = jnp.where(col // OUT_DIM == row, 1.0, 0.0).astype(jnp.float32)
    as_ref[...] = jnp.dot(h * atts_ref[...], m, preferred_element_type=jnp.float32)
    ad_ref[...] = jnp.dot(h * attd_ref[...], m, preferred_element_type=jnp.float32)


def _proj(x, wt, atts, attd):
    grid = (N_NODES // BN1,)
    return pl.pallas_call(
        _proj_body,
        grid=grid,
        in_specs=[
            pl.BlockSpec((BN1, IN_DIM), lambda i: (i, 0)),
            pl.BlockSpec((IN_DIM, HC), lambda i: (0, 0)),
            pl.BlockSpec((1, HC), lambda i: (0, 0)),
            pl.BlockSpec((1, HC), lambda i: (0, 0)),
        ],
        out_specs=[
            pl.BlockSpec((BN1, HC), lambda i: (i, 0)),
            pl.BlockSpec((BN1, HEADS), lambda i: (i, 0)),
            pl.BlockSpec((BN1, HEADS), lambda i: (i, 0)),
        ],
        out_shape=[
            jax.ShapeDtypeStruct((N_NODES, HC), jnp.float32),
            jax.ShapeDtypeStruct((N_NODES, HEADS), jnp.float32),
            jax.ShapeDtypeStruct((N_NODES, HEADS), jnp.float32),
        ],
    )(x, wt, atts, attd)


def _make_edge_kernel():
    mesh = plsc.VectorSubcoreMesh(core_axis_name="c", subcore_axis_name="s")

    @functools.partial(
        pl.kernel,
        out_type=jax.ShapeDtypeStruct((NC, NPAD, HEXT), jnp.float32),
        mesh=mesh,
        compiler_params=pltpu.CompilerParams(use_tc_tiling_on_sc=False),
        scratch_types=[
            pltpu.VMEM((2, 2, SB), jnp.int32),
            pltpu.VMEM((2, 2, SB), jnp.int32),
            pltpu.VMEM((2, 2, SB), jnp.int32),
            pltpu.VMEM((2, 2, SB), jnp.int32),
            pltpu.VMEM((EB, HEXT), jnp.float32),
            pltpu.VMEM((EB, HEXT), jnp.float32),
            pltpu.VMEM((EB, DEN_W), jnp.float32),
            pltpu.VMEM((EB, DEN_W), jnp.float32),
            pltpu.VMEM((SB, HEXT), jnp.float32),
            pltpu.VMEM((SB, HEXT), jnp.float32),
            pltpu.VMEM_SHARED((NPAD, HEXT), jnp.float32),
            pltpu.SemaphoreType.DMA,
            pltpu.SemaphoreType.DMA,
            pltpu.SemaphoreType.DMA,
            pltpu.SemaphoreType.DMA,
            pltpu.SemaphoreType.DMA,
            pltpu.SemaphoreType.DMA,
        ],
    )
    def edge_kernel(hext_hbm, adp_hbm, sd_hbm, acc_hbm,
                    sdr0, sdr1, sdr2, sdr3, hg0, hg1, adg0, adg1,
                    msg0, msg1, acc,
                    sh0, sh1, sa0, sa1, sm0, sm1):
        c = lax.axis_index("c")
        s = lax.axis_index("s")
        wid = c * NS + s
        sdr = (sdr0, sdr1, sdr2, sdr3)
        gs = ((hg0, adg0, sh0, sa0), (hg1, adg1, sh1, sa1))
        msgs = ((msg0, sm0), (msg1, sm1))

        def _zero(i, _):
            r = i // (HEXT // L)
            col = (i % (HEXT // L)) * L
            hg0[r, pl.ds(col, L)] = jnp.zeros((L,), jnp.float32)
            return 0

        lax.fori_loop(0, RCH * (HEXT // L), _zero, 0)

        def _init_chunk(k, _):
            r0 = s * RPT + k * RCH
            pltpu.sync_copy(hg0.at[pl.ds(0, RCH)], acc.at[pl.ds(r0, RCH)])
            return 0

        lax.fori_loop(0, RPT // RCH, _init_chunk, 0)
        plsc.subcore_barrier()

        def _gfire(blk, g, sd):
            hg, adg, sh, sa = g
            pltpu.sync_copy(sd_hbm.at[:, wid, blk], sd)
            pltpu.async_copy(hext_hbm.at[sd.at[0, 0]], hg.at[pl.ds(0, SB)], sh)
            pltpu.async_copy(hext_hbm.at[sd.at[0, 1]], hg.at[pl.ds(SB, SB)], sh)
            pltpu.async_copy(adp_hbm.at[sd.at[1, 0]], adg.at[pl.ds(0, SB)], sa)
            pltpu.async_copy(adp_hbm.at[sd.at[1, 1]], adg.at[pl.ds(SB, SB)], sa)

        def _gwait(g, sd):
            hg, adg, sh, sa = g
            pltpu.make_async_copy(hext_hbm.at[sd.at[0, 0]], hg.at[pl.ds(0, SB)], sh).wait()
            pltpu.make_async_copy(hext_hbm.at[sd.at[0, 1]], hg.at[pl.ds(SB, SB)], sh).wait()
            pltpu.make_async_copy(adp_hbm.at[sd.at[1, 0]], adg.at[pl.ds(0, SB)], sa).wait()
            pltpu.make_async_copy(adp_hbm.at[sd.at[1, 1]], adg.at[pl.ds(SB, SB)], sa).wait()

        def _compute_sub(g, m, sub):
            hg, adg, sh, sa = g
            msg, sm = m
            base = sub * SB

            def _edge(e, _):
                ee = base + e
                a = hg[ee, pl.ds(HC, L)] + adg[ee, :]
                w = jnp.exp(jnp.maximum(a, 0.2 * a))
                msg[e, pl.ds(HC, L)] = w
                for hh in range(HEADS):
                    msg[e, pl.ds(hh * L, L)] = hg[ee, pl.ds(hh * L, L)] * w[hh]
                return 0

            lax.fori_loop(0, SB, _edge, 0)

        def _sfire(m, sd, sub):
            msg, sm = m
            pltpu.async_copy(msg, acc.at[sd.at[1, sub]], sm, add=True)

        def _swait(m, sd, sub):
            msg, sm = m
            pltpu.make_async_copy(msg, acc.at[sd.at[1, sub]], sm).wait()

        def _step(sd_cur, sd_prev, sd_next, g, gn, fire_next, first):
            if first:
                _gfire(fire_next, gn, sd_next)
            else:
                pl.when(fire_next < NBLK)(
                    lambda: _gfire(fire_next, gn, sd_next))
            _gwait(g, sd_cur)
            if not first:
                _swait(msgs[0], sd_prev, 0)
            _compute_sub(g, msgs[0], 0)
            _sfire(msgs[0], sd_cur, 0)
            if not first:
                _swait(msgs[1], sd_prev, 1)
            _compute_sub(g, msgs[1], 1)
            _sfire(msgs[1], sd_cur, 1)

        _gfire(0, gs[0], sdr[0])
        _step(sdr[0], None, sdr[1], gs[0], gs[1], 1, True)

        def _quad(i, _):
            b = 4 * i + 1
            _step(sdr[1], sdr[0], sdr[2], gs[1], gs[0], b + 1, False)
            _step(sdr[2], sdr[1], sdr[3], gs[0], gs[1], b + 2, False)
            _step(sdr[3], sdr[2], sdr[0], gs[1], gs[0], b + 3, False)
            _step(sdr[0], sdr[3], sdr[1], gs[0], gs[1], b + 4, False)
            return 0

        lax.fori_loop(0, (NBLK - 1) // 4, _quad, 0)
        _swait(msgs[0], sdr[0], 0)
        _swait(msgs[1], sdr[0], 1)
        plsc.subcore_barrier()

        def _drain_chunk(k, _):
            r0 = s * RPT + k * RCH
            pltpu.sync_copy(acc.at[pl.ds(r0, RCH)], hg0.at[pl.ds(0, RCH)])
            pltpu.sync_copy(hg0.at[pl.ds(0, RCH)], acc_hbm.at[c, pl.ds(r0, RCH)])
            return 0

        lax.fori_loop(0, RPT // RCH, _drain_chunk, 0)

    return edge_kernel


_edge_kernel = _make_edge_kernel()


def _combine_body(acc_ref, x_ref, bias_ref, out_ref):
    num = acc_ref[0, :, 0:HC] + acc_ref[1, :, 0:HC]
    den = acc_ref[0, :, HC:HC + HEADS] + acc_ref[1, :, HC:HC + HEADS]
    inv = 1.0 / jnp.maximum(den, 1e-30)
    col = lax.broadcasted_iota(jnp.int32, (HEADS, HC), 1)
    row = lax.broadcasted_iota(jnp.int32, (HEADS, HC), 0)
    m = jnp.where(col // OUT_DIM == row, 1.0, 0.0).astype(jnp.float32)
    inv128 = jnp.dot(inv, m, preferred_element_type=jnp.float32)
    out = jnp.maximum(num * inv128 + bias_ref[...], 0.0)
    out_ref[...] = x_ref[...] + out


def _combine(acc2, x, bias2):
    grid = (N_NODES // BN1,)
    return pl.pallas_call(
        _combine_body,
        grid=grid,
        in_specs=[
            pl.BlockSpec((NC, BN1, HEXT), lambda i: (0, i, 0)),
            pl.BlockSpec((BN1, IN_DIM), lambda i: (i, 0)),
            pl.BlockSpec((1, HC), lambda i: (0, 0)),
        ],
        out_specs=pl.BlockSpec((BN1, HC), lambda i: (i, 0)),
        out_shape=jax.ShapeDtypeStruct((N_NODES, HC), jnp.float32),
    )(acc2, x, bias2)


def kernel(x, edge_index, W, att_src, att_dst, bias):
    src = edge_index[0].astype(jnp.int32).reshape(NC * NS, NBLK, 2, SB)
    dst = edge_index[1].astype(jnp.int32).reshape(NC * NS, NBLK, 2, SB)
    sd = jnp.stack([src, dst])
    wt = W.T
    atts = att_src.reshape(1, HC)
    attd = att_dst.reshape(1, HC)
    h, a_s, a_d = _proj(x, wt, atts, attd)
    pad8 = jnp.zeros((N_NODES, 8), jnp.float32)
    hext = jnp.concatenate([h, a_s, pad8], axis=1)
    adp = jnp.concatenate([a_d, pad8], axis=1)
    acc2 = _edge_kernel(hext, adp, sd)
    acc2 = acc2[:, :N_NODES]
    return _combine(acc2, x, bias.reshape(1, HC))


# speedup vs baseline: 1.7543x; 1.0098x over previous
import functools

import jax
import jax.numpy as jnp
from jax import lax
from jax.experimental import pallas as pl
from jax.experimental.pallas import tpu as pltpu
from jax.experimental.pallas import tpu_sc as plsc

N_NODES = 10000
N_EDGES = 320000
IN_DIM = 128
OUT_DIM = 16
HEADS = 8
HC = HEADS * OUT_DIM

NC = 2
NS = 16
L = 16

EPT = N_EDGES // (NC * NS)
EB = 80
SB = EB // 2
NBLK = EPT // EB
HEXT = 144
DEN_W = 16
NPAD = 10240
RPT = NPAD // NS
RCH = 40

BN1 = 2000


def _proj_body(x_ref, wt_ref, atts_ref, attd_ref, h_ref, as_ref, ad_ref):
    h = jnp.dot(x_ref[...], wt_ref[...], preferred_element_type=jnp.float32)
    h_ref[...] = h
    col = lax.broadcasted_iota(jnp.int32, (HC, HEADS), 0)
    row = lax.broadcasted_iota(jnp.int32, (HC, HEADS), 1)
    m = jnp.where(col // OUT_DIM == row, 1.0, 0.0).astype(jnp.float32)
    as_ref[...] = jnp.dot(h * atts_ref[...], m, preferred_element_type=jnp.float32)
    ad_ref[...] = jnp.dot(h * attd_ref[...], m, preferred_element_type=jnp.float32)


def _proj(x, wt, atts, attd):
    grid = (N_NODES // BN1,)
    return pl.pallas_call(
        _proj_body,
        grid=grid,
        in_specs=[
            pl.BlockSpec((BN1, IN_DIM), lambda i: (i, 0)),
            pl.BlockSpec((IN_DIM, HC), lambda i: (0, 0)),
            pl.BlockSpec((1, HC), lambda i: (0, 0)),
            pl.BlockSpec((1, HC), lambda i: (0, 0)),
        ],
        out_specs=[
            pl.BlockSpec((BN1, HC), lambda i: (i, 0)),
            pl.BlockSpec((BN1, HEADS), lambda i: (i, 0)),
            pl.BlockSpec((BN1, HEADS), lambda i: (i, 0)),
        ],
        out_shape=[
            jax.ShapeDtypeStruct((N_NODES, HC), jnp.float32),
            jax.ShapeDtypeStruct((N_NODES, HEADS), jnp.float32),
            jax.ShapeDtypeStruct((N_NODES, HEADS), jnp.float32),
        ],
    )(x, wt, atts, attd)


def _make_edge_kernel():
    mesh = plsc.VectorSubcoreMesh(core_axis_name="c", subcore_axis_name="s")

    @functools.partial(
        pl.kernel,
        out_type=jax.ShapeDtypeStruct((NC, NPAD, HEXT), jnp.float32),
        mesh=mesh,
        compiler_params=pltpu.CompilerParams(use_tc_tiling_on_sc=False),
        scratch_types=[
            pltpu.VMEM((2, 2, SB), jnp.int32),
            pltpu.VMEM((2, 2, SB), jnp.int32),
            pltpu.VMEM((2, 2, SB), jnp.int32),
            pltpu.VMEM((2, 2, SB), jnp.int32),
            pltpu.VMEM((EB, HEXT), jnp.float32),
            pltpu.VMEM((EB, HEXT), jnp.float32),
            pltpu.VMEM((EB, DEN_W), jnp.float32),
            pltpu.VMEM((EB, DEN_W), jnp.float32),
            pltpu.VMEM((SB, HEXT), jnp.float32),
            pltpu.VMEM((SB, HEXT), jnp.float32),
            pltpu.VMEM_SHARED((NPAD, HEXT), jnp.float32),
            pltpu.SemaphoreType.DMA,
            pltpu.SemaphoreType.DMA,
            pltpu.SemaphoreType.DMA,
            pltpu.SemaphoreType.DMA,
            pltpu.SemaphoreType.DMA,
            pltpu.SemaphoreType.DMA,
        ],
    )
    def edge_kernel(hext_hbm, adp_hbm, sd_hbm, acc_hbm,
                    sdr0, sdr1, sdr2, sdr3, hg0, hg1, adg0, adg1,
                    msg0, msg1, acc,
                    sh0, sh1, sa0, sa1, sm0, sm1):
        c = lax.axis_index("c")
        s = lax.axis_index("s")
        wid = c * NS + s
        sdr = (sdr0, sdr1, sdr2, sdr3)
        gs = ((hg0, adg0, sh0, sa0), (hg1, adg1, sh1, sa1))
        msgs = ((msg0, sm0), (msg1, sm1))

        def _zero(i, _):
            r = i // (HEXT // L)
            col = (i % (HEXT // L)) * L
            hg0[r, pl.ds(col, L)] = jnp.zeros((L,), jnp.float32)
            return 0

        lax.fori_loop(0, RCH * (HEXT // L), _zero, 0)

        def _init_fire(k, _):
            r0 = s * RPT + k * RCH
            pltpu.async_copy(hg0.at[pl.ds(0, RCH)], acc.at[pl.ds(r0, RCH)], sh0)
            return 0

        lax.fori_loop(0, RPT // RCH, _init_fire, 0)

        def _init_wait(k, _):
            r0 = s * RPT + k * RCH
            pltpu.make_async_copy(hg0.at[pl.ds(0, RCH)], acc.at[pl.ds(r0, RCH)], sh0).wait()
            return 0

        lax.fori_loop(0, RPT // RCH, _init_wait, 0)
        plsc.subcore_barrier()

        def _gfire(blk, g, sd):
            hg, adg, sh, sa = g
            pltpu.sync_copy(sd_hbm.at[:, wid, blk], sd)
            pltpu.async_copy(hext_hbm.at[sd.at[0, 0]], hg.at[pl.ds(0, SB)], sh)
            pltpu.async_copy(hext_hbm.at[sd.at[0, 1]], hg.at[pl.ds(SB, SB)], sh)
            pltpu.async_copy(adp_hbm.at[sd.at[1, 0]], adg.at[pl.ds(0, SB)], sa)
            pltpu.async_copy(adp_hbm.at[sd.at[1, 1]], adg.at[pl.ds(SB, SB)], sa)

        def _gwait(g, sd):
            hg, adg, sh, sa = g
            pltpu.make_async_copy(hext_hbm.at[sd.at[0, 0]], hg.at[pl.ds(0, SB)], sh).wait()
            pltpu.make_async_copy(hext_hbm.at[sd.at[0, 1]], hg.at[pl.ds(SB, SB)], sh).wait()
            pltpu.make_async_copy(adp_hbm.at[sd.at[1, 0]], adg.at[pl.ds(0, SB)], sa).wait()
            pltpu.make_async_copy(adp_hbm.at[sd.at[1, 1]], adg.at[pl.ds(SB, SB)], sa).wait()

        def _compute_sub(g, m, sub):
            hg, adg, sh, sa = g
            msg, sm = m
            base = sub * SB

            def _edge(e, _):
                ee = base + e
                a = hg[ee, pl.ds(HC, L)] + adg[ee, :]
                w = jnp.exp(jnp.maximum(a, 0.2 * a))
                msg[e, pl.ds(HC, L)] = w
                for hh in range(HEADS):
                    msg[e, pl.ds(hh * L, L)] = hg[ee, pl.ds(hh * L, L)] * w[hh]
                return 0

            lax.fori_loop(0, SB, _edge, 0)

        def _sfire(m, sd, sub):
            msg, sm = m
            pltpu.async_copy(msg, acc.at[sd.at[1, sub]], sm, add=True)

        def _swait(m, sd, sub):
            msg, sm = m
            pltpu.make_async_copy(msg, acc.at[sd.at[1, sub]], sm).wait()

        def _step(sd_cur, sd_prev, sd_next, g, gn, fire_next, first):
            if first:
                _gfire(fire_next, gn, sd_next)
            else:
                pl.when(fire_next < NBLK)(
                    lambda: _gfire(fire_next, gn, sd_next))
            _gwait(g, sd_cur)
            if not first:
                _swait(msgs[0], sd_prev, 0)
            _compute_sub(g, msgs[0], 0)
            _sfire(msgs[0], sd_cur, 0)
            if not first:
                _swait(msgs[1], sd_prev, 1)
            _compute_sub(g, msgs[1], 1)
            _sfire(msgs[1], sd_cur, 1)

        _gfire(0, gs[0], sdr[0])
        _step(sdr[0], None, sdr[1], gs[0], gs[1], 1, True)

        def _quad(i, _):
            b = 4 * i + 1
            _step(sdr[1], sdr[0], sdr[2], gs[1], gs[0], b + 1, False)
            _step(sdr[2], sdr[1], sdr[3], gs[0], gs[1], b + 2, False)
            _step(sdr[3], sdr[2], sdr[0], gs[1], gs[0], b + 3, False)
            _step(sdr[0], sdr[3], sdr[1], gs[0], gs[1], b + 4, False)
            return 0

        lax.fori_loop(0, (NBLK - 1) // 4, _quad, 0)
        _swait(msgs[0], sdr[0], 0)
        _swait(msgs[1], sdr[0], 1)
        plsc.subcore_barrier()

        def _dr_r0(k):
            return s * RPT + k * RCH

        def _d_in_fire(k, m, sem):
            pltpu.async_copy(acc.at[pl.ds(_dr_r0(k), RCH)], m, sem)

        def _d_in_wait(k, m, sem):
            pltpu.make_async_copy(acc.at[pl.ds(_dr_r0(k), RCH)], m, sem).wait()

        def _d_out_fire(k, m, sem):
            pltpu.async_copy(m, acc_hbm.at[c, pl.ds(_dr_r0(k), RCH)], sem)

        def _d_out_wait(k, m, sem):
            pltpu.make_async_copy(m, acc_hbm.at[c, pl.ds(_dr_r0(k), RCH)], sem).wait()

        _d_in_fire(0, msg0, sa0)

        def _d_pair(j, _):
            k = 2 * j
            _d_in_wait(k, msg0, sa0)
            pl.when(j > 0)(lambda: _d_out_wait(k - 1, msg1, sm1))
            _d_in_fire(k + 1, msg1, sa1)
            _d_out_fire(k, msg0, sm0)
            _d_in_wait(k + 1, msg1, sa1)
            _d_out_wait(k, msg0, sm0)
            pl.when(k + 2 < RPT // RCH)(lambda: _d_in_fire(k + 2, msg0, sa0))
            _d_out_fire(k + 1, msg1, sm1)
            return 0

        lax.fori_loop(0, (RPT // RCH) // 2, _d_pair, 0)
        _d_out_wait(RPT // RCH - 1, msg1, sm1)

    return edge_kernel


_edge_kernel = _make_edge_kernel()


def _combine_body(acc_ref, x_ref, bias_ref, out_ref):
    num = acc_ref[0, :, 0:HC] + acc_ref[1, :, 0:HC]
    den = acc_ref[0, :, HC:HC + HEADS] + acc_ref[1, :, HC:HC + HEADS]
    inv = 1.0 / jnp.maximum(den, 1e-30)
    col = lax.broadcasted_iota(jnp.int32, (HEADS, HC), 1)
    row = lax.broadcasted_iota(jnp.int32, (HEADS, HC), 0)
    m = jnp.where(col // OUT_DIM == row, 1.0, 0.0).astype(jnp.float32)
    inv128 = jnp.dot(inv, m, preferred_element_type=jnp.float32)
    out = jnp.maximum(num * inv128 + bias_ref[...], 0.0)
    out_ref[...] = x_ref[...] + out


def _combine(acc2, x, bias2):
    grid = (N_NODES // BN1,)
    return pl.pallas_call(
        _combine_body,
        grid=grid,
        in_specs=[
            pl.BlockSpec((NC, BN1, HEXT), lambda i: (0, i, 0)),
            pl.BlockSpec((BN1, IN_DIM), lambda i: (i, 0)),
            pl.BlockSpec((1, HC), lambda i: (0, 0)),
        ],
        out_specs=pl.BlockSpec((BN1, HC), lambda i: (i, 0)),
        out_shape=jax.ShapeDtypeStruct((N_NODES, HC), jnp.float32),
    )(acc2, x, bias2)


def kernel(x, edge_index, W, att_src, att_dst, bias):
    src = edge_index[0].astype(jnp.int32).reshape(NC * NS, NBLK, 2, SB)
    dst = edge_index[1].astype(jnp.int32).reshape(NC * NS, NBLK, 2, SB)
    sd = jnp.stack([src, dst])
    wt = W.T
    atts = att_src.reshape(1, HC)
    attd = att_dst.reshape(1, HC)
    h, a_s, a_d = _proj(x, wt, atts, attd)
    pad8 = jnp.zeros((N_NODES, 8), jnp.float32)
    hext = jnp.concatenate([h, a_s, pad8], axis=1)
    adp = jnp.concatenate([a_d, pad8], axis=1)
    acc2 = _edge_kernel(hext, adp, sd)
    acc2 = acc2[:, :N_NODES]
    return _combine(acc2, x, bias.reshape(1, HC))
